# baseline (speedup 1.0000x reference)
import jax
import jax.numpy as jnp
from jax.experimental import pallas as pl
from jax.experimental.pallas import tpu as pltpu

EMBED = 768
HID = 1536
PATCH = 16
IMG = 384
HP = IMG // PATCH
NPATCH = HP * HP
TM = 512


def _mlp_block_kernel(fi_ref, fm_ref, keep_ref,
                      W1i_ref, W2i_ref, W1c_ref, W2c_ref,
                      b1i_ref, b2i_ref, b1c_ref, b2c_ref,
                      oi_ref, oc_ref):
    x = fi_ref[...]
    xm = x + fm_ref[...]
    keep = keep_ref[...]

    hi = jnp.dot(xm, W1i_ref[...], preferred_element_type=jnp.float32)
    hi = jnp.maximum(hi + b1i_ref[...], 0.0).astype(jnp.bfloat16)
    oi = jnp.dot(hi, W2i_ref[...], preferred_element_type=jnp.float32)
    oi_ref[...] = (oi + b2i_ref[...]) * keep

    hc = jnp.dot(x, W1c_ref[...], preferred_element_type=jnp.float32)
    hc = jnp.maximum(hc + b1c_ref[...], 0.0).astype(jnp.bfloat16)
    oc = jnp.dot(hc, W2c_ref[...], preferred_element_type=jnp.float32)
    oc_ref[...] = (oc + b2c_ref[...]) * keep


def _fold(p, b):
    p = p.reshape(b, HP, HP, 3, PATCH, PATCH)
    p = jnp.transpose(p, (0, 3, 1, 4, 2, 5))
    return p.reshape(b, 3, IMG, IMG)


def kernel(f_id, f_mod, img, mod, mask, W1i, b1i, W2i, b2i, W1c, b1c, W2c, b2c):
    b = f_id.shape[0]
    ntok = b * NPATCH
    fi = f_id[:, 1:, :].reshape(ntok, EMBED).astype(jnp.bfloat16)
    fm = f_mod[:, 1:, :].reshape(ntok, EMBED).astype(jnp.bfloat16)
    keep = (~mask[:, 1:]).reshape(ntok, 1).astype(jnp.float32)

    w1i = W1i.astype(jnp.bfloat16)
    w2i = W2i.astype(jnp.bfloat16)
    w1c = W1c.astype(jnp.bfloat16)
    w2c = W2c.astype(jnp.bfloat16)

    grid = (ntok // TM,)
    row_spec = pl.BlockSpec((TM, EMBED), lambda i: (i, 0))
    keep_spec = pl.BlockSpec((TM, 1), lambda i: (i, 0))
    w1_spec = pl.BlockSpec((EMBED, HID), lambda i: (0, 0))
    w2_spec = pl.BlockSpec((HID, EMBED), lambda i: (0, 0))
    b1_spec = pl.BlockSpec((1, HID), lambda i: (0, 0))
    b2_spec = pl.BlockSpec((1, EMBED), lambda i: (0, 0))
    out_spec = pl.BlockSpec((TM, EMBED), lambda i: (i, 0))

    oi, oc = pl.pallas_call(
        _mlp_block_kernel,
        grid=grid,
        in_specs=[row_spec, row_spec, keep_spec,
                  w1_spec, w2_spec, w1_spec, w2_spec,
                  b1_spec, b2_spec, b1_spec, b2_spec],
        out_specs=[out_spec, out_spec],
        out_shape=[jax.ShapeDtypeStruct((ntok, EMBED), jnp.float32),
                   jax.ShapeDtypeStruct((ntok, EMBED), jnp.float32)],
        compiler_params=pltpu.CompilerParams(
            dimension_semantics=("arbitrary",)),
    )(fi, fm, keep, w1i, w2i, w1c, w2c,
      b1i.reshape(1, HID), b2i.reshape(1, EMBED),
      b1c.reshape(1, HID), b2c.reshape(1, EMBED))

    recon_intra = _fold(oi.reshape(b, NPATCH, EMBED), b)
    recon_cross = _fold(oc.reshape(b, NPATCH, EMBED), b)
    return (recon_intra, recon_cross)


# speedup vs baseline: 2.1128x; 2.1128x over previous
import jax
import jax.numpy as jnp
from jax.experimental import pallas as pl
from jax.experimental.pallas import tpu as pltpu

EMBED = 768
HID = 1536
PATCH = 16
IMG = 384
HP = IMG // PATCH
NPATCH = HP * HP


def _fold_block(p):
    p = p.reshape(HP, HP, 3, PATCH, PATCH)
    p = jnp.transpose(p, (2, 0, 3, 1, 4))
    return p.reshape(3, IMG, IMG)


def _image_kernel(fi_ref, fm_ref, keep_ref,
                  W1i_ref, W2i_ref, W1c_ref, W2c_ref,
                  b1i_ref, b2i_ref, b1c_ref, b2c_ref,
                  oi_ref, oc_ref):
    x = fi_ref[0, 1:, :].astype(jnp.bfloat16)
    xm = x + fm_ref[0, 1:, :].astype(jnp.bfloat16)
    keep = keep_ref[0]

    hi = jnp.dot(xm, W1i_ref[...], preferred_element_type=jnp.float32)
    hi = jnp.maximum(hi + b1i_ref[...], 0.0).astype(jnp.bfloat16)
    oi = jnp.dot(hi, W2i_ref[...], preferred_element_type=jnp.float32)
    oi = (oi + b2i_ref[...]) * keep
    oi_ref[0] = _fold_block(oi)

    hc = jnp.dot(x, W1c_ref[...], preferred_element_type=jnp.float32)
    hc = jnp.maximum(hc + b1c_ref[...], 0.0).astype(jnp.bfloat16)
    oc = jnp.dot(hc, W2c_ref[...], preferred_element_type=jnp.float32)
    oc = (oc + b2c_ref[...]) * keep
    oc_ref[0] = _fold_block(oc)


def kernel(f_id, f_mod, img, mod, mask, W1i, b1i, W2i, b2i, W1c, b1c, W2c, b2c):
    b = f_id.shape[0]
    keep = (~mask[:, 1:]).astype(jnp.float32).reshape(b, NPATCH, 1)

    w1i = W1i.astype(jnp.bfloat16)
    w2i = W2i.astype(jnp.bfloat16)
    w1c = W1c.astype(jnp.bfloat16)
    w2c = W2c.astype(jnp.bfloat16)

    tok_spec = pl.BlockSpec((1, NPATCH + 1, EMBED), lambda i: (i, 0, 0))
    keep_spec = pl.BlockSpec((1, NPATCH, 1), lambda i: (i, 0, 0))
    w1_spec = pl.BlockSpec((EMBED, HID), lambda i: (0, 0))
    w2_spec = pl.BlockSpec((HID, EMBED), lambda i: (0, 0))
    b1_spec = pl.BlockSpec((1, HID), lambda i: (0, 0))
    b2_spec = pl.BlockSpec((1, EMBED), lambda i: (0, 0))
    out_spec = pl.BlockSpec((1, 3, IMG, IMG), lambda i: (i, 0, 0, 0))

    recon_intra, recon_cross = pl.pallas_call(
        _image_kernel,
        grid=(b,),
        in_specs=[tok_spec, tok_spec, keep_spec,
                  w1_spec, w2_spec, w1_spec, w2_spec,
                  b1_spec, b2_spec, b1_spec, b2_spec],
        out_specs=[out_spec, out_spec],
        out_shape=[jax.ShapeDtypeStruct((b, 3, IMG, IMG), jnp.float32),
                   jax.ShapeDtypeStruct((b, 3, IMG, IMG), jnp.float32)],
        compiler_params=pltpu.CompilerParams(
            dimension_semantics=("arbitrary",)),
    )(f_id, f_mod, keep, w1i, w2i, w1c, w2c,
      b1i.reshape(1, HID), b2i.reshape(1, EMBED),
      b1c.reshape(1, HID), b2c.reshape(1, EMBED))

    return (recon_intra, recon_cross)


# speedup vs baseline: 2.7421x; 1.2979x over previous
import functools

import jax
import jax.numpy as jnp
from jax import lax
from jax.experimental import pallas as pl
from jax.experimental.pallas import tpu as pltpu
from jax.experimental.pallas import tpu_sc as plsc

EMBED = 768
HID = 1536
PATCH = 16
IMG = 384
HP = IMG // PATCH
NPATCH = HP * HP
NCORES = 2
NSUB = 16
NW = NCORES * NSUB


def _mlp_kernel(fi_ref, fm_ref, keep_ref,
                W1i_ref, W2i_ref, W1c_ref, W2c_ref,
                b1i_ref, b2i_ref, b1c_ref, b2c_ref,
                oi_ref, oc_ref):
    x = fi_ref[0, 1:, :].astype(jnp.bfloat16)
    xm = x + fm_ref[0, 1:, :].astype(jnp.bfloat16)
    keep = keep_ref[0]

    hi = jnp.dot(xm, W1i_ref[...], preferred_element_type=jnp.float32)
    hi = jnp.maximum(hi + b1i_ref[...], 0.0).astype(jnp.bfloat16)
    oi = jnp.dot(hi, W2i_ref[...], preferred_element_type=jnp.float32)
    oi_ref[0] = (oi + b2i_ref[...]) * keep

    hc = jnp.dot(x, W1c_ref[...], preferred_element_type=jnp.float32)
    hc = jnp.maximum(hc + b1c_ref[...], 0.0).astype(jnp.bfloat16)
    oc = jnp.dot(hc, W2c_ref[...], preferred_element_type=jnp.float32)
    oc_ref[0] = (oc + b2c_ref[...]) * keep


def _run_mlps(f_id, f_mod, keep, w1i, w2i, w1c, w2c, b1i, b2i, b1c, b2c):
    b = f_id.shape[0]
    tok_spec = pl.BlockSpec((1, NPATCH + 1, EMBED), lambda i: (i, 0, 0))
    keep_spec = pl.BlockSpec((1, NPATCH, 1), lambda i: (i, 0, 0))
    w1_spec = pl.BlockSpec((EMBED, HID), lambda i: (0, 0))
    w2_spec = pl.BlockSpec((HID, EMBED), lambda i: (0, 0))
    b1_spec = pl.BlockSpec((1, HID), lambda i: (0, 0))
    b2_spec = pl.BlockSpec((1, EMBED), lambda i: (0, 0))
    out_spec = pl.BlockSpec((1, NPATCH, EMBED), lambda i: (i, 0, 0))

    return pl.pallas_call(
        _mlp_kernel,
        grid=(b,),
        in_specs=[tok_spec, tok_spec, keep_spec,
                  w1_spec, w2_spec, w1_spec, w2_spec,
                  b1_spec, b2_spec, b1_spec, b2_spec],
        out_specs=[out_spec, out_spec],
        out_shape=[jax.ShapeDtypeStruct((b, NPATCH, EMBED), jnp.float32),
                   jax.ShapeDtypeStruct((b, NPATCH, EMBED), jnp.float32)],
        compiler_params=pltpu.CompilerParams(
            dimension_semantics=("arbitrary",)),
    )(f_id, f_mod, keep, w1i, w2i, w1c, w2c, b1i, b2i, b1c, b2c)


def _fold_body(src, dst, blk, inbuf, outbuf):
    bb = blk // HP
    h = blk % HP
    pltpu.sync_copy(src.at[bb, pl.ds(h * HP, HP), :], inbuf)

    def cr_body(cr, carry):
        c = cr // PATCH
        r = cr % PATCH
        for w in range(HP):
            v = inbuf[w, pl.ds(c * 256 + r * PATCH, PATCH)]
            outbuf[c, r, pl.ds(w * PATCH, PATCH)] = v
        return carry

    lax.fori_loop(0, 3 * PATCH, cr_body, 0)
    pltpu.sync_copy(outbuf, dst.at[bb, :, pl.ds(h * PATCH, PATCH), :])


def _fold_sc_kernel(nblocks, pi_hbm, pc_hbm, oi_hbm, oc_hbm, inbuf, outbuf):
    wid = lax.axis_index("core") * NSUB + lax.axis_index("subcore")
    per_w = nblocks // NW

    def body_i(t, carry):
        _fold_body(pi_hbm, oi_hbm, t * NW + wid, inbuf, outbuf)
        return carry

    def body_c(t, carry):
        _fold_body(pc_hbm, oc_hbm, t * NW + wid, inbuf, outbuf)
        return carry

    lax.fori_loop(0, per_w, body_i, 0)
    lax.fori_loop(0, per_w, body_c, 0)


def _run_fold(p_intra, p_cross):
    b = p_intra.shape[0]
    nblocks = b * HP
    mesh = plsc.VectorSubcoreMesh(core_axis_name="core",
                                  subcore_axis_name="subcore")
    out_sd = jax.ShapeDtypeStruct((b, 3, IMG, IMG), jnp.float32)
    fold = pl.kernel(
        functools.partial(_fold_sc_kernel, nblocks),
        out_type=[out_sd, out_sd],
        mesh=mesh,
        scratch_types=[pltpu.VMEM((HP, EMBED), jnp.float32),
                       pltpu.VMEM((3, PATCH, IMG), jnp.float32)],
    )
    return fold(p_intra, p_cross)


def kernel(f_id, f_mod, img, mod, mask, W1i, b1i, W2i, b2i, W1c, b1c, W2c, b2c):
    b = f_id.shape[0]
    keep = (~mask[:, 1:]).astype(jnp.float32).reshape(b, NPATCH, 1)

    p_intra, p_cross = _run_mlps(
        f_id, f_mod, keep,
        W1i.astype(jnp.bfloat16), W2i.astype(jnp.bfloat16),
        W1c.astype(jnp.bfloat16), W2c.astype(jnp.bfloat16),
        b1i.reshape(1, HID), b2i.reshape(1, EMBED),
        b1c.reshape(1, HID), b2c.reshape(1, EMBED))

    recon_intra, recon_cross = _run_fold(p_intra, p_cross)
    return (recon_intra, recon_cross)


# speedup vs baseline: 3.3201x; 1.2108x over previous
import functools

import jax
import jax.numpy as jnp
from jax import lax
from jax.experimental import pallas as pl
from jax.experimental.pallas import tpu as pltpu
from jax.experimental.pallas import tpu_sc as plsc

EMBED = 768
HID = 1536
PATCH = 16
IMG = 384
HP = IMG // PATCH
NPATCH = HP * HP
NCORES = 2
NSUB = 16
NW = NCORES * NSUB


def _mlp_kernel(fi_ref, fm_ref, keep_ref,
                W1i_ref, W2i_ref, W1c_ref, W2c_ref,
                b1i_ref, b2i_ref, b1c_ref, b2c_ref,
                oi_ref, oc_ref):
    x = fi_ref[0, 1:, :].astype(jnp.bfloat16)
    xm = x + fm_ref[0, 1:, :].astype(jnp.bfloat16)
    keep = keep_ref[0]

    hi = jnp.dot(xm, W1i_ref[...], preferred_element_type=jnp.float32)
    hi = jnp.maximum(hi + b1i_ref[...], 0.0).astype(jnp.bfloat16)
    oi = jnp.dot(hi, W2i_ref[...], preferred_element_type=jnp.float32)
    oi_ref[0] = (oi + b2i_ref[...]) * keep

    hc = jnp.dot(x, W1c_ref[...], preferred_element_type=jnp.float32)
    hc = jnp.maximum(hc + b1c_ref[...], 0.0).astype(jnp.bfloat16)
    oc = jnp.dot(hc, W2c_ref[...], preferred_element_type=jnp.float32)
    oc_ref[0] = (oc + b2c_ref[...]) * keep


def _run_mlps(f_id, f_mod, keep, w1i, w2i, w1c, w2c, b1i, b2i, b1c, b2c):
    b = f_id.shape[0]
    tok_spec = pl.BlockSpec((1, NPATCH + 1, EMBED), lambda i: (i, 0, 0))
    keep_spec = pl.BlockSpec((1, NPATCH, 1), lambda i: (i, 0, 0))
    w1_spec = pl.BlockSpec((EMBED, HID), lambda i: (0, 0))
    w2_spec = pl.BlockSpec((HID, EMBED), lambda i: (0, 0))
    b1_spec = pl.BlockSpec((1, HID), lambda i: (0, 0))
    b2_spec = pl.BlockSpec((1, EMBED), lambda i: (0, 0))
    out_spec = pl.BlockSpec((1, NPATCH, EMBED), lambda i: (i, 0, 0))

    return pl.pallas_call(
        _mlp_kernel,
        grid=(b,),
        in_specs=[tok_spec, tok_spec, keep_spec,
                  w1_spec, w2_spec, w1_spec, w2_spec,
                  b1_spec, b2_spec, b1_spec, b2_spec],
        out_specs=[out_spec, out_spec],
        out_shape=[jax.ShapeDtypeStruct((b, NPATCH, EMBED), jnp.float32),
                   jax.ShapeDtypeStruct((b, NPATCH, EMBED), jnp.float32)],
        compiler_params=pltpu.CompilerParams(
            dimension_semantics=("arbitrary",)),
    )(f_id, f_mod, keep, w1i, w2i, w1c, w2c, b1i, b2i, b1c, b2c)


def _reorder_block(inbuf, outbuf):
    def w_body(w, carry):
        for k in range(3 * PATCH):
            v = inbuf[w, pl.ds(k * PATCH, PATCH)]
            outbuf[k // PATCH, k % PATCH, pl.ds(w * PATCH, PATCH)] = v
        return carry

    lax.fori_loop(0, HP, w_body, 0)


def _fold_one_array(src, dst, wid, per_w, bufs):
    (in0, in1, ob0, ob1, isem0, isem1, osem0, osem1) = bufs
    ins = (in0, in1)
    obs = (ob0, ob1)
    isems = (isem0, isem1)
    osems = (osem0, osem1)

    def src_block(t):
        blk = t * NW + wid
        return src.at[blk // HP, pl.ds((blk % HP) * HP, HP), :]

    def dst_block(t):
        blk = t * NW + wid
        return dst.at[blk // HP, :, pl.ds((blk % HP) * PATCH, PATCH), :]

    pltpu.make_async_copy(src_block(0), in0, isem0).start()
    pltpu.make_async_copy(src_block(1), in1, isem1).start()

    def pair_body(i, carry):
        for s in range(2):
            t = i * 2 + s
            pltpu.make_async_copy(src_block(t), ins[s], isems[s]).wait()

            @pl.when(i >= 1)
            def _():
                pltpu.make_async_copy(obs[s], dst_block(t - 2), osems[s]).wait()

            _reorder_block(ins[s], obs[s])

            @pl.when(i < per_w // 2 - 1)
            def _():
                pltpu.make_async_copy(src_block(t + 2), ins[s], isems[s]).start()

            pltpu.make_async_copy(obs[s], dst_block(t), osems[s]).start()
        return carry

    lax.fori_loop(0, per_w // 2, pair_body, 0)
    last = per_w - 2
    pltpu.make_async_copy(ob0, dst_block(last), osem0).wait()
    pltpu.make_async_copy(ob1, dst_block(last + 1), osem1).wait()


def _fold_sc_kernel(nblocks, pi_hbm, pc_hbm, oi_hbm, oc_hbm, *bufs):
    wid = lax.axis_index("core") * NSUB + lax.axis_index("subcore")
    per_w = nblocks // NW
    _fold_one_array(pi_hbm, oi_hbm, wid, per_w, bufs)
    _fold_one_array(pc_hbm, oc_hbm, wid, per_w, bufs)


def _run_fold(p_intra, p_cross):
    b = p_intra.shape[0]
    nblocks = b * HP
    mesh = plsc.VectorSubcoreMesh(core_axis_name="core",
                                  subcore_axis_name="subcore")
    out_sd = jax.ShapeDtypeStruct((b, 3, IMG, IMG), jnp.float32)
    fold = pl.kernel(
        functools.partial(_fold_sc_kernel, nblocks),
        out_type=[out_sd, out_sd],
        mesh=mesh,
        scratch_types=[pltpu.VMEM((HP, EMBED), jnp.float32),
                       pltpu.VMEM((HP, EMBED), jnp.float32),
                       pltpu.VMEM((3, PATCH, IMG), jnp.float32),
                       pltpu.VMEM((3, PATCH, IMG), jnp.float32),
                       pltpu.SemaphoreType.DMA,
                       pltpu.SemaphoreType.DMA,
                       pltpu.SemaphoreType.DMA,
                       pltpu.SemaphoreType.DMA],
    )
    return fold(p_intra, p_cross)


def kernel(f_id, f_mod, img, mod, mask, W1i, b1i, W2i, b2i, W1c, b1c, W2c, b2c):
    b = f_id.shape[0]
    keep = (~mask[:, 1:]).astype(jnp.float32).reshape(b, NPATCH, 1)

    p_intra, p_cross = _run_mlps(
        f_id, f_mod, keep,
        W1i.astype(jnp.bfloat16), W2i.astype(jnp.bfloat16),
        W1c.astype(jnp.bfloat16), W2c.astype(jnp.bfloat16),
        b1i.reshape(1, HID), b2i.reshape(1, EMBED),
        b1c.reshape(1, HID), b2c.reshape(1, EMBED))

    recon_intra, recon_cross = _run_fold(p_intra, p_cross)
    return (recon_intra, recon_cross)


# speedup vs baseline: 3.3554x; 1.0106x over previous
import functools

import jax
import jax.numpy as jnp
from jax import lax
from jax.experimental import pallas as pl
from jax.experimental.pallas import tpu as pltpu
from jax.experimental.pallas import tpu_sc as plsc

EMBED = 768
HID = 1536
PATCH = 16
IMG = 384
HP = IMG // PATCH
NPATCH = HP * HP
NCORES = 2
NSUB = 16
NW = NCORES * NSUB


MI = 2


def _mlp_kernel(fi_ref, fm_ref, keep_ref,
                W1i_ref, W2i_ref, W1c_ref, W2c_ref,
                b1i_ref, b2i_ref, b1c_ref, b2c_ref,
                oi_ref, oc_ref):
    x = fi_ref[:, 1:, :].astype(jnp.bfloat16).reshape(MI * NPATCH, EMBED)
    xm = x + fm_ref[:, 1:, :].astype(jnp.bfloat16).reshape(MI * NPATCH, EMBED)
    keep = keep_ref[...].reshape(MI * NPATCH, 1)

    hi = jnp.dot(xm, W1i_ref[...], preferred_element_type=jnp.float32)
    hi = jnp.maximum(hi + b1i_ref[...], 0.0).astype(jnp.bfloat16)
    oi = jnp.dot(hi, W2i_ref[...], preferred_element_type=jnp.float32)
    oi_ref[...] = ((oi + b2i_ref[...]) * keep).reshape(MI, NPATCH, EMBED)

    hc = jnp.dot(x, W1c_ref[...], preferred_element_type=jnp.float32)
    hc = jnp.maximum(hc + b1c_ref[...], 0.0).astype(jnp.bfloat16)
    oc = jnp.dot(hc, W2c_ref[...], preferred_element_type=jnp.float32)
    oc_ref[...] = ((oc + b2c_ref[...]) * keep).reshape(MI, NPATCH, EMBED)


def _run_mlps(f_id, f_mod, keep, w1i, w2i, w1c, w2c, b1i, b2i, b1c, b2c):
    b = f_id.shape[0]
    tok_spec = pl.BlockSpec((MI, NPATCH + 1, EMBED), lambda i: (i, 0, 0))
    keep_spec = pl.BlockSpec((MI, NPATCH, 1), lambda i: (i, 0, 0))
    w1_spec = pl.BlockSpec((EMBED, HID), lambda i: (0, 0))
    w2_spec = pl.BlockSpec((HID, EMBED), lambda i: (0, 0))
    b1_spec = pl.BlockSpec((1, HID), lambda i: (0, 0))
    b2_spec = pl.BlockSpec((1, EMBED), lambda i: (0, 0))
    out_spec = pl.BlockSpec((MI, NPATCH, EMBED), lambda i: (i, 0, 0))

    return pl.pallas_call(
        _mlp_kernel,
        grid=(b // MI,),
        in_specs=[tok_spec, tok_spec, keep_spec,
                  w1_spec, w2_spec, w1_spec, w2_spec,
                  b1_spec, b2_spec, b1_spec, b2_spec],
        out_specs=[out_spec, out_spec],
        out_shape=[jax.ShapeDtypeStruct((b, NPATCH, EMBED), jnp.float32),
                   jax.ShapeDtypeStruct((b, NPATCH, EMBED), jnp.float32)],
        compiler_params=pltpu.CompilerParams(
            dimension_semantics=("arbitrary",)),
    )(f_id, f_mod, keep, w1i, w2i, w1c, w2c, b1i, b2i, b1c, b2c)


def _reorder_block(inbuf, outbuf):
    def w_body(w, carry):
        for k in range(3 * PATCH):
            v = inbuf[w, pl.ds(k * PATCH, PATCH)]
            outbuf[k // PATCH, k % PATCH, pl.ds(w * PATCH, PATCH)] = v
        return carry

    lax.fori_loop(0, HP, w_body, 0)


def _fold_one_array(src, dst, wid, per_w, bufs):
    (in0, in1, ob0, ob1, isem0, isem1, osem0, osem1) = bufs
    ins = (in0, in1)
    obs = (ob0, ob1)
    isems = (isem0, isem1)
    osems = (osem0, osem1)

    def src_block(t):
        blk = t * NW + wid
        return src.at[blk // HP, pl.ds((blk % HP) * HP, HP), :]

    def dst_block(t):
        blk = t * NW + wid
        return dst.at[blk // HP, :, pl.ds((blk % HP) * PATCH, PATCH), :]

    pltpu.make_async_copy(src_block(0), in0, isem0).start()
    pltpu.make_async_copy(src_block(1), in1, isem1).start()

    def pair_body(i, carry):
        for s in range(2):
            t = i * 2 + s
            pltpu.make_async_copy(src_block(t), ins[s], isems[s]).wait()

            @pl.when(i >= 1)
            def _():
                pltpu.make_async_copy(obs[s], dst_block(t - 2), osems[s]).wait()

            _reorder_block(ins[s], obs[s])

            @pl.when(i < per_w // 2 - 1)
            def _():
                pltpu.make_async_copy(src_block(t + 2), ins[s], isems[s]).start()

            pltpu.make_async_copy(obs[s], dst_block(t), osems[s]).start()
        return carry

    lax.fori_loop(0, per_w // 2, pair_body, 0)
    last = per_w - 2
    pltpu.make_async_copy(ob0, dst_block(last), osem0).wait()
    pltpu.make_async_copy(ob1, dst_block(last + 1), osem1).wait()


def _fold_sc_kernel(nblocks, pi_hbm, pc_hbm, oi_hbm, oc_hbm, *bufs):
    wid = lax.axis_index("core") * NSUB + lax.axis_index("subcore")
    per_w = nblocks // NW
    _fold_one_array(pi_hbm, oi_hbm, wid, per_w, bufs)
    _fold_one_array(pc_hbm, oc_hbm, wid, per_w, bufs)


def _run_fold(p_intra, p_cross):
    b = p_intra.shape[0]
    nblocks = b * HP
    mesh = plsc.VectorSubcoreMesh(core_axis_name="core",
                                  subcore_axis_name="subcore")
    out_sd = jax.ShapeDtypeStruct((b, 3, IMG, IMG), jnp.float32)
    fold = pl.kernel(
        functools.partial(_fold_sc_kernel, nblocks),
        out_type=[out_sd, out_sd],
        mesh=mesh,
        scratch_types=[pltpu.VMEM((HP, EMBED), jnp.float32),
                       pltpu.VMEM((HP, EMBED), jnp.float32),
                       pltpu.VMEM((3, PATCH, IMG), jnp.float32),
                       pltpu.VMEM((3, PATCH, IMG), jnp.float32),
                       pltpu.SemaphoreType.DMA,
                       pltpu.SemaphoreType.DMA,
                       pltpu.SemaphoreType.DMA,
                       pltpu.SemaphoreType.DMA],
    )
    return fold(p_intra, p_cross)


def kernel(f_id, f_mod, img, mod, mask, W1i, b1i, W2i, b2i, W1c, b1c, W2c, b2c):
    b = f_id.shape[0]
    keep = (~mask[:, 1:]).astype(jnp.float32).reshape(b, NPATCH, 1)

    p_intra, p_cross = _run_mlps(
        f_id, f_mod, keep,
        W1i.astype(jnp.bfloat16), W2i.astype(jnp.bfloat16),
        W1c.astype(jnp.bfloat16), W2c.astype(jnp.bfloat16),
        b1i.reshape(1, HID), b2i.reshape(1, EMBED),
        b1c.reshape(1, HID), b2c.reshape(1, EMBED))

    recon_intra, recon_cross = _run_fold(p_intra, p_cross)
    return (recon_intra, recon_cross)


# speedup vs baseline: 3.6965x; 1.1017x over previous
import functools

import jax
import jax.numpy as jnp
from jax import lax
from jax.experimental import pallas as pl
from jax.experimental.pallas import tpu as pltpu
from jax.experimental.pallas import tpu_sc as plsc

EMBED = 768
HID = 1536
PATCH = 16
IMG = 384
HP = IMG // PATCH
NPATCH = HP * HP
NCORES = 2
NSUB = 16
NW = NCORES * NSUB


MI = 2


def _mlp_kernel(fi_ref, fm_ref, keep_ref,
                W1i_ref, W2i_ref, W1c_ref, W2c_ref,
                b1i_ref, b2i_ref, b1c_ref, b2c_ref,
                oi_ref, oc_ref):
    x = fi_ref[:, 1:, :].astype(jnp.bfloat16).reshape(MI * NPATCH, EMBED)
    xm = x + fm_ref[:, 1:, :].astype(jnp.bfloat16).reshape(MI * NPATCH, EMBED)
    keep = keep_ref[...].reshape(MI * NPATCH, 1)

    hi = jnp.dot(xm, W1i_ref[...], preferred_element_type=jnp.float32)
    hi = jnp.maximum(hi + b1i_ref[...], 0.0).astype(jnp.bfloat16)
    oi = jnp.dot(hi, W2i_ref[...], preferred_element_type=jnp.float32)
    oi_ref[...] = ((oi + b2i_ref[...]) * keep).reshape(MI, NPATCH, EMBED)

    hc = jnp.dot(x, W1c_ref[...], preferred_element_type=jnp.float32)
    hc = jnp.maximum(hc + b1c_ref[...], 0.0).astype(jnp.bfloat16)
    oc = jnp.dot(hc, W2c_ref[...], preferred_element_type=jnp.float32)
    oc_ref[...] = ((oc + b2c_ref[...]) * keep).reshape(MI, NPATCH, EMBED)


def _run_mlps(f_id, f_mod, keep, w1i, w2i, w1c, w2c, b1i, b2i, b1c, b2c,
              nimg, img0):
    off = img0 // MI
    tok_spec = pl.BlockSpec((MI, NPATCH + 1, EMBED), lambda i: (off + i, 0, 0))
    keep_spec = pl.BlockSpec((MI, NPATCH, 1), lambda i: (off + i, 0, 0))
    w1_spec = pl.BlockSpec((EMBED, HID), lambda i: (0, 0))
    w2_spec = pl.BlockSpec((HID, EMBED), lambda i: (0, 0))
    b1_spec = pl.BlockSpec((1, HID), lambda i: (0, 0))
    b2_spec = pl.BlockSpec((1, EMBED), lambda i: (0, 0))
    out_spec = pl.BlockSpec((MI, NPATCH, EMBED), lambda i: (i, 0, 0))

    return pl.pallas_call(
        _mlp_kernel,
        grid=(nimg // MI,),
        in_specs=[tok_spec, tok_spec, keep_spec,
                  w1_spec, w2_spec, w1_spec, w2_spec,
                  b1_spec, b2_spec, b1_spec, b2_spec],
        out_specs=[out_spec, out_spec],
        out_shape=[jax.ShapeDtypeStruct((nimg, NPATCH, EMBED), jnp.float32),
                   jax.ShapeDtypeStruct((nimg, NPATCH, EMBED), jnp.float32)],
        compiler_params=pltpu.CompilerParams(
            dimension_semantics=("arbitrary",)),
    )(f_id, f_mod, keep, w1i, w2i, w1c, w2c, b1i, b2i, b1c, b2c)


def _reorder_block(inbuf, outbuf):
    def w_body(w, carry):
        for k in range(3 * PATCH):
            v = inbuf[w, pl.ds(k * PATCH, PATCH)]
            outbuf[k // PATCH, k % PATCH, pl.ds(w * PATCH, PATCH)] = v
        return carry

    lax.fori_loop(0, HP, w_body, 0)


def _fold_one_array(src, dst, wid, per_w, bufs):
    (in0, in1, ob0, ob1, isem0, isem1, osem0, osem1) = bufs
    ins = (in0, in1)
    obs = (ob0, ob1)
    isems = (isem0, isem1)
    osems = (osem0, osem1)

    def src_block(t):
        blk = t * NW + wid
        return src.at[blk // HP, pl.ds((blk % HP) * HP, HP), :]

    def dst_block(t):
        blk = t * NW + wid
        return dst.at[blk // HP, :, pl.ds((blk % HP) * PATCH, PATCH), :]

    pltpu.make_async_copy(src_block(0), in0, isem0).start()
    pltpu.make_async_copy(src_block(1), in1, isem1).start()

    def pair_body(i, carry):
        for s in range(2):
            t = i * 2 + s
            pltpu.make_async_copy(src_block(t), ins[s], isems[s]).wait()

            @pl.when(i >= 1)
            def _():
                pltpu.make_async_copy(obs[s], dst_block(t - 2), osems[s]).wait()

            _reorder_block(ins[s], obs[s])

            @pl.when(i < per_w // 2 - 1)
            def _():
                pltpu.make_async_copy(src_block(t + 2), ins[s], isems[s]).start()

            pltpu.make_async_copy(obs[s], dst_block(t), osems[s]).start()
        return carry

    lax.fori_loop(0, per_w // 2, pair_body, 0)
    last = per_w - 2
    pltpu.make_async_copy(ob0, dst_block(last), osem0).wait()
    pltpu.make_async_copy(ob1, dst_block(last + 1), osem1).wait()


def _fold_sc_kernel(nblocks, pi_hbm, pc_hbm, oi_hbm, oc_hbm, *bufs):
    wid = lax.axis_index("core") * NSUB + lax.axis_index("subcore")
    per_w = nblocks // NW
    _fold_one_array(pi_hbm, oi_hbm, wid, per_w, bufs)
    _fold_one_array(pc_hbm, oc_hbm, wid, per_w, bufs)


def _run_fold(p_intra, p_cross):
    b = p_intra.shape[0]
    nblocks = b * HP
    mesh = plsc.VectorSubcoreMesh(core_axis_name="core",
                                  subcore_axis_name="subcore")
    out_sd = jax.ShapeDtypeStruct((b, 3, IMG, IMG), jnp.float32)
    fold = pl.kernel(
        functools.partial(_fold_sc_kernel, nblocks),
        out_type=[out_sd, out_sd],
        mesh=mesh,
        scratch_types=[pltpu.VMEM((HP, EMBED), jnp.float32),
                       pltpu.VMEM((HP, EMBED), jnp.float32),
                       pltpu.VMEM((3, PATCH, IMG), jnp.float32),
                       pltpu.VMEM((3, PATCH, IMG), jnp.float32),
                       pltpu.SemaphoreType.DMA,
                       pltpu.SemaphoreType.DMA,
                       pltpu.SemaphoreType.DMA,
                       pltpu.SemaphoreType.DMA],
    )
    return fold(p_intra, p_cross)


NCHUNK = 8


def kernel(f_id, f_mod, img, mod, mask, W1i, b1i, W2i, b2i, W1c, b1c, W2c, b2c):
    b = f_id.shape[0]
    keep = (~mask[:, 1:]).astype(jnp.float32).reshape(b, NPATCH, 1)

    w1i = W1i.astype(jnp.bfloat16)
    w2i = W2i.astype(jnp.bfloat16)
    w1c = W1c.astype(jnp.bfloat16)
    w2c = W2c.astype(jnp.bfloat16)
    b1i2 = b1i.reshape(1, HID)
    b2i2 = b2i.reshape(1, EMBED)
    b1c2 = b1c.reshape(1, HID)
    b2c2 = b2c.reshape(1, EMBED)

    cb = b // NCHUNK
    folds = []
    for c in range(NCHUNK):
        p_intra, p_cross = _run_mlps(
            f_id, f_mod, keep,
            w1i, w2i, w1c, w2c, b1i2, b2i2, b1c2, b2c2,
            nimg=cb, img0=c * cb)
        folds.append(_run_fold(p_intra, p_cross))

    recon_intra = jnp.concatenate([f[0] for f in folds], axis=0)
    recon_cross = jnp.concatenate([f[1] for f in folds], axis=0)
    return (recon_intra, recon_cross)
